# baseline (speedup 1.0000x reference)
import functools
import jax
import jax.numpy as jnp
from jax import lax
from jax.experimental import pallas as pl
from jax.experimental.pallas import tpu as pltpu
from jax.experimental.pallas import tpu_sc as plsc

NC, NS = 2, 16
NW = NC * NS
NBUF = 4
LANES = 16


def kernel(x, pos_table):
    B, S, D = x.shape
    seq_per_w = S // NW
    vregs_per_row = D // LANES
    mesh = plsc.VectorSubcoreMesh(core_axis_name="c", subcore_axis_name="s")

    half = seq_per_w // 2
    NP = 2 * B

    @functools.partial(
        pl.kernel,
        out_type=jax.ShapeDtypeStruct((B, S, D), jnp.float32),
        mesh=mesh,
        scratch_types=[
            pltpu.VMEM((seq_per_w, D), jnp.float32),
            pltpu.VMEM((NBUF, half, D), jnp.float32),
            pltpu.SemaphoreType.DMA,
            pltpu.SemaphoreType.DMA((NBUF,)),
            pltpu.SemaphoreType.DMA((NBUF,)),
        ],
    )
    def sc_add(x_hbm, pos_hbm, out_hbm, pos_v, bufs, sem_p, sem_in, sem_out):
        wid = lax.axis_index("s") * NC + lax.axis_index("c")
        s0 = wid * seq_per_w
        pos_desc = pltpu.async_copy(pos_hbm.at[pl.ds(s0, seq_per_w)], pos_v, sem_p)

        def in_copy(p):
            b = p % NBUF
            return pltpu.async_copy(
                x_hbm.at[p // 2, pl.ds(s0 + (p % 2) * half, half)],
                bufs.at[b], sem_in.at[b])

        def out_copy(p):
            b = p % NBUF
            return pltpu.async_copy(
                bufs.at[b], out_hbm.at[p // 2, pl.ds(s0 + (p % 2) * half, half)],
                sem_out.at[b])

        descs = {("in", 0): in_copy(0), ("in", 1): in_copy(1)}
        pos_desc.wait()
        rows_per_it = 8
        for p in range(NP):
            b = p % NBUF
            descs[("in", p)].wait()

            def add_rows(g, _, b=b, p=p):
                r0 = g * rows_per_it
                pr0 = (p % 2) * half
                for dr in range(rows_per_it):
                    for k in range(vregs_per_row):
                        plsc.addupdate(
                            bufs.at[b, r0 + dr, pl.ds(k * LANES, LANES)],
                            pos_v[pr0 + r0 + dr, pl.ds(k * LANES, LANES)])
                return 0
            lax.fori_loop(0, half // rows_per_it, add_rows, 0)

            descs[("out", p)] = out_copy(p)
            if p + 2 < NP:
                if p + 2 - NBUF >= 0:
                    descs[("out", p + 2 - NBUF)].wait()
                descs[("in", p + 2)] = in_copy(p + 2)
        for p in range(NP - NBUF, NP):
            descs[("out", p)].wait()

    return sc_add(x, pos_table)


# speedup vs baseline: 1.0600x; 1.0600x over previous
import functools
import jax
import jax.numpy as jnp
from jax import lax
from jax.experimental import pallas as pl
from jax.experimental.pallas import tpu as pltpu
from jax.experimental.pallas import tpu_sc as plsc

NC, NS = 2, 16
NW = NC * NS
NBUF = 2
LANES = 16


def kernel(x, pos_table):
    B, S, D = x.shape
    seq_per_w = S // NW
    vregs_per_row = D // LANES
    mesh = plsc.VectorSubcoreMesh(core_axis_name="c", subcore_axis_name="s")

    @functools.partial(
        pl.kernel,
        out_type=jax.ShapeDtypeStruct((B, S, D), jnp.float32),
        mesh=mesh,
        scratch_types=[
            pltpu.VMEM((seq_per_w, D), jnp.float32),
            pltpu.VMEM((NBUF, seq_per_w, D), jnp.float32),
            pltpu.SemaphoreType.DMA,
            pltpu.SemaphoreType.DMA((NBUF,)),
            pltpu.SemaphoreType.DMA((NBUF,)),
        ],
    )
    def sc_add(x_hbm, pos_hbm, out_hbm, pos_v, bufs, sem_p, sem_in, sem_out):
        wid = lax.axis_index("s") * NC + lax.axis_index("c")
        s0 = wid * seq_per_w
        pos_desc = pltpu.async_copy(pos_hbm.at[pl.ds(s0, seq_per_w)], pos_v, sem_p)

        def in_copy(c):
            b = c % NBUF
            return pltpu.async_copy(
                x_hbm.at[c, pl.ds(s0, seq_per_w)], bufs.at[b], sem_in.at[b])

        def out_copy(c):
            b = c % NBUF
            return pltpu.async_copy(
                bufs.at[b], out_hbm.at[c, pl.ds(s0, seq_per_w)], sem_out.at[b])

        descs = {("in", 0): in_copy(0)}
        pos_desc.wait()
        rows_per_it = 8
        for c in range(B):
            b = c % NBUF
            if c + 1 < B:
                if c >= 1:
                    descs[("out", c - 1)].wait()
                descs[("in", c + 1)] = in_copy(c + 1)
            descs[("in", c)].wait()

            def add_rows(g, _, b=b):
                r0 = g * rows_per_it
                for dr in range(rows_per_it):
                    for k in range(vregs_per_row):
                        plsc.addupdate(
                            bufs.at[b, r0 + dr, pl.ds(k * LANES, LANES)],
                            pos_v[r0 + dr, pl.ds(k * LANES, LANES)])
                return 0
            lax.fori_loop(0, seq_per_w // rows_per_it, add_rows, 0)

            descs[("out", c)] = out_copy(c)
        descs[("out", B - 2)].wait()
        descs[("out", B - 1)].wait()

    return sc_add(x, pos_table)
